# Initial kernel scaffold; baseline (speedup 1.0000x reference)
import jax
import jax.numpy as jnp
from jax.experimental import pallas as pl


def kernel(x, router_logits, w3_w1_weight, w2_weight):
    raise NotImplementedError("write your pallas kernel here")


# speedup vs baseline: 1.7898x; 1.7898x over previous
import jax
import jax.numpy as jnp
from jax.experimental import pallas as pl
from jax.experimental.pallas import tpu as pltpu

NUM_EXPERTS = 8
TOP_K = 2
HIDDEN = 1024
INTER = 1024
TOKENS = 2048
TM = 512


def _moe_body(logits_ref, x_ref, w31_ref, w2_ref, out_ref, combine_ref):
    e = pl.program_id(1)

    @pl.when(e == 0)
    def _():
        logits = logits_ref[...]
        m = jnp.max(logits, axis=-1, keepdims=True)
        ex = jnp.exp(logits - m)
        probs = ex / jnp.sum(ex, axis=-1, keepdims=True)
        lanes = jax.lax.broadcasted_iota(jnp.int32, probs.shape, 1)
        m1 = jnp.max(probs, axis=-1, keepdims=True)
        i1 = jnp.min(jnp.where(probs == m1, lanes, NUM_EXPERTS), axis=-1, keepdims=True)
        is1 = lanes == i1
        masked = jnp.where(is1, -jnp.inf, probs)
        m2 = jnp.max(masked, axis=-1, keepdims=True)
        i2 = jnp.min(jnp.where(masked == m2, lanes, NUM_EXPERTS), axis=-1, keepdims=True)
        is2 = lanes == i2
        denom = m1 + m2
        combine_ref[...] = (jnp.where(is1, m1, 0.0) + jnp.where(is2, m2, 0.0)) / denom

    xs = x_ref[...]
    proj = jax.lax.dot_general(
        xs, w31_ref[0], (((1,), (1,)), ((), ())), preferred_element_type=jnp.float32
    )
    up = proj[:, :INTER]
    gate = proj[:, INTER:]
    h = gate * jax.nn.sigmoid(gate) * up
    y = jax.lax.dot_general(
        h, w2_ref[0], (((1,), (1,)), ((), ())), preferred_element_type=jnp.float32
    )
    lanes = jax.lax.broadcasted_iota(jnp.int32, (xs.shape[0], NUM_EXPERTS), 1)
    w = jnp.sum(jnp.where(lanes == e, combine_ref[...], 0.0), axis=-1, keepdims=True)

    @pl.when(e == 0)
    def _():
        out_ref[...] = w * y

    @pl.when(e != 0)
    def _():
        out_ref[...] = out_ref[...] + w * y


def kernel(x, router_logits, w3_w1_weight, w2_weight):
    grid = (TOKENS // TM, NUM_EXPERTS)
    out = pl.pallas_call(
        _moe_body,
        grid=grid,
        in_specs=[
            pl.BlockSpec((TM, NUM_EXPERTS), lambda i, e: (i, 0)),
            pl.BlockSpec((TM, HIDDEN), lambda i, e: (i, 0)),
            pl.BlockSpec((1, 2 * INTER, HIDDEN), lambda i, e: (e, 0, 0)),
            pl.BlockSpec((1, HIDDEN, INTER), lambda i, e: (e, 0, 0)),
        ],
        out_specs=pl.BlockSpec((TM, HIDDEN), lambda i, e: (i, 0)),
        out_shape=jax.ShapeDtypeStruct((TOKENS, HIDDEN), jnp.float32),
        scratch_shapes=[pltpu.VMEM((TM, NUM_EXPERTS), jnp.float32)],
        compiler_params=pltpu.CompilerParams(
            dimension_semantics=("parallel", "arbitrary")
        ),
    )(router_logits, x, w3_w1_weight, w2_weight)
    return out.astype(x.dtype)
